# baseline (speedup 1.0000x reference)
import functools
import math

import numpy as np

import jax
import jax.numpy as jnp
from jax.experimental import pallas as pl
from jax.experimental.pallas import tpu as pltpu

F32 = jnp.float32
BF16 = jnp.bfloat16
NEG = -1e30
LN_THETA = math.log(10000.0)
EPS = 1e-5

_h = np.arange(8)[:, None] * 64 + np.arange(32)[None, :]
_qx1 = _h.reshape(256)
_QKV_PERM = np.concatenate([_qx1, _qx1 + 32, _qx1 + 512, _qx1 + 544,
                            np.arange(1024, 1536)])


def _mm(a, b):
    return jax.lax.dot_general(
        a.astype(BF16), b.astype(BF16), (((1,), (0,)), ((), ())),
        preferred_element_type=F32)


def _mm_t(a, b, out_dtype=F32):
    r = jax.lax.dot_general(
        a.astype(BF16), b.astype(BF16), (((1,), (1,)), ((), ())),
        preferred_element_type=F32)
    return r.astype(out_dtype)


def _rms(x, w):
    return x * jax.lax.rsqrt(jnp.mean(x * x, axis=-1, keepdims=True) + EPS) * w


def _softmax_rows(sc):
    e = jnp.exp(sc)
    s = jnp.sum(e, axis=-1, keepdims=True, dtype=F32)
    return e * (1.0 / s).astype(sc.dtype)


def _tables_body(enc_ref, rt_ref):
    j = jax.lax.broadcasted_iota(jnp.int32, (2048, 32), 1).astype(F32)
    inv = jnp.exp(j * (-LN_THETA / 32.0))
    pos = jax.lax.broadcasted_iota(jnp.int32, (2048, 32), 0)
    f = pos.astype(F32) * inv
    enc_ref[:, :32] = jnp.cos(f)
    enc_ref[:, 32:] = jnp.sin(f)
    t136 = jnp.concatenate([jnp.cos(f[:136]), jnp.sin(f[:136])], axis=-1)
    for i in range(16):
        rt_ref[i * 136:(i + 1) * 136, :] = t136


def _tables_call():
    return pl.pallas_call(
        _tables_body,
        out_specs=[pl.BlockSpec((2048, 64), lambda: (0, 0)),
                   pl.BlockSpec((2176, 64), lambda: (0, 0))],
        out_shape=[jax.ShapeDtypeStruct((2048, 64), F32),
                   jax.ShapeDtypeStruct((2176, 64), F32)],
    )()


def _qkv_body(x_ref, nw_ref, w_ref, tbl_ref, o_ref):
    xn = _rms(x_ref[...], nw_ref[...])
    qkv = _mm(xn, w_ref[...])
    c32 = tbl_ref[:, :32]
    s32 = tbl_ref[:, 32:]
    c = jnp.concatenate([c32, c32, c32, c32], axis=-1)
    s = jnp.concatenate([s32, s32, s32, s32], axis=-1)
    c = jnp.concatenate([c, c], axis=-1)
    s = jnp.concatenate([s, s], axis=-1)
    for base, cc, ss in ((0, c * 0.125, s * 0.125), (512, c, s)):
        a = qkv[:, base:base + 256]
        b = qkv[:, base + 256:base + 512]
        o_ref[:, base:base + 256] = (a * cc + b * ss).astype(BF16)
        o_ref[:, base + 256:base + 512] = (b * cc - a * ss).astype(BF16)
    o_ref[:, 1024:] = qkv[:, 1024:].astype(BF16)


def _qkv_call(x, nw, w, tbl, tiles):
    n, d = x.shape
    dout = w.shape[1]
    tn = n // tiles
    return pl.pallas_call(
        _qkv_body,
        grid=(tiles,),
        in_specs=[
            pl.BlockSpec((tn, d), lambda i: (i, 0)),
            pl.BlockSpec((1, d), lambda i: (0, 0)),
            pl.BlockSpec((d, dout), lambda i: (0, 0)),
            pl.BlockSpec((tn, 64), lambda i: (i, 0)),
        ],
        out_specs=pl.BlockSpec((tn, dout), lambda i: (i, 0)),
        out_shape=jax.ShapeDtypeStruct((n, dout), BF16),
        compiler_params=pltpu.CompilerParams(
            dimension_semantics=("parallel",)),
    )(x, nw, w, tbl)


def _enc_attn_body(doc_c_ref, doc_r_ref, z_ref, o_ref):
    step = pl.program_id(0)
    neg = jnp.asarray(NEG, BF16)
    for s in range(4):
        @pl.when(step == s)
        def _(s=s):
            qt = (3 * s) % 4
            kw = (qt + 1) * 512
            row = qt * 512
            rb = (row + jax.lax.broadcasted_iota(jnp.int32, (512, 1), 0)) // 128
            cb = jax.lax.broadcasted_iota(jnp.int32, (1, kw), 1) // 128
            mask = (rb >= cb) & (doc_c_ref[...] == doc_r_ref[0:1, 0:kw])
            outs = []
            for h in range(8):
                q = jnp.concatenate(
                    [z_ref[row:row + 512, h * 32:(h + 1) * 32],
                     z_ref[row:row + 512, 256 + h * 32:256 + (h + 1) * 32]],
                    axis=-1)
                k = jnp.concatenate(
                    [z_ref[0:kw, 512 + h * 32:512 + (h + 1) * 32],
                     z_ref[0:kw, 768 + h * 32:768 + (h + 1) * 32]], axis=-1)
                v = z_ref[0:kw, 1024 + h * 64:1024 + (h + 1) * 64]
                sc = _mm_t(q, k, out_dtype=BF16)
                sc = jnp.where(mask, sc, neg)
                p = _softmax_rows(sc)
                outs.append(_mm(p, v).astype(BF16))
            o_ref[...] = jnp.concatenate(outs, axis=-1)


def _enc_attn_call(doc_c, doc_r, qkv):
    return pl.pallas_call(
        _enc_attn_body,
        grid=(4,),
        in_specs=[
            pl.BlockSpec((512, 1), lambda s: ((3 * s) % 4, 0)),
            pl.BlockSpec((1, 2048), lambda s: (0, 0)),
            pl.BlockSpec((2048, 1536), lambda s: (0, 0)),
        ],
        out_specs=pl.BlockSpec((512, 512), lambda s: ((3 * s) % 4, 0)),
        out_shape=jax.ShapeDtypeStruct((2048, 512), BF16),
        compiler_params=pltpu.CompilerParams(
            dimension_semantics=("parallel",)),
    )(doc_c, doc_r, qkv)


def _rt_fused_body(x_ref, nw_ref, w_ref, tbl_ref, o_ref):
    xn = _rms(x_ref[...], nw_ref[...])
    qkv = _mm(xn, w_ref[...])
    c32 = tbl_ref[:, :32]
    s32 = tbl_ref[:, 32:]
    c = jnp.concatenate([c32, c32, c32, c32], axis=-1)
    s = jnp.concatenate([s32, s32, s32, s32], axis=-1)
    c = jnp.concatenate([c, c], axis=-1)
    s = jnp.concatenate([s, s], axis=-1)
    rot = []
    for base, cc, ss in ((0, c * 0.125, s * 0.125), (512, c, s)):
        a = qkv[:, base:base + 256]
        b = qkv[:, base + 256:base + 512]
        rot.append((a * cc + b * ss).astype(BF16))
        rot.append((b * cc - a * ss).astype(BF16))
    qa, qb, ka, kb = rot
    v = qkv[:, 1024:].astype(BF16)
    kmask = jax.lax.broadcasted_iota(jnp.int32, (1, 136), 1) < 132
    neg = jnp.asarray(NEG, BF16)
    blocks = []
    for blk in range(4):
        r = slice(blk * 136, (blk + 1) * 136)
        outs = []
        for h in range(8):
            q = jnp.concatenate([qa[r, h * 32:(h + 1) * 32],
                                 qb[r, h * 32:(h + 1) * 32]], axis=-1)
            k = jnp.concatenate([ka[r, h * 32:(h + 1) * 32],
                                 kb[r, h * 32:(h + 1) * 32]], axis=-1)
            sc = _mm_t(q, k, out_dtype=BF16)
            sc = jnp.where(kmask, sc, neg)
            p = _softmax_rows(sc)
            outs.append(_mm(p, v[r, h * 64:(h + 1) * 64]).astype(BF16))
        blocks.append(jnp.concatenate(outs, axis=-1))
    o_ref[...] = jnp.concatenate(blocks, axis=0)


def _rt_fused_call(x, nw, w, tbl):
    return pl.pallas_call(
        _rt_fused_body,
        grid=(4,),
        in_specs=[
            pl.BlockSpec((544, 512), lambda i: (i, 0)),
            pl.BlockSpec((1, 512), lambda i: (0, 0)),
            pl.BlockSpec((512, 1536), lambda i: (0, 0)),
            pl.BlockSpec((544, 64), lambda i: (i, 0)),
        ],
        out_specs=pl.BlockSpec((544, 512), lambda i: (i, 0)),
        out_shape=jax.ShapeDtypeStruct((2176, 512), BF16),
        compiler_params=pltpu.CompilerParams(
            dimension_semantics=("parallel",)),
    )(x, nw, w, tbl)


def _rt_attn_body(z_ref, o_ref):
    kmask = jax.lax.broadcasted_iota(jnp.int32, (1, 136), 1) < 132
    neg = jnp.asarray(NEG, BF16)
    for blk in range(2):
        z = z_ref[blk]
        outs = []
        for h in range(8):
            q = jnp.concatenate([z[:, h * 32:(h + 1) * 32],
                                 z[:, 256 + h * 32:256 + (h + 1) * 32]], axis=-1)
            k = jnp.concatenate([z[:, 512 + h * 32:512 + (h + 1) * 32],
                                 z[:, 768 + h * 32:768 + (h + 1) * 32]], axis=-1)
            v = z[:, 1024 + h * 64:1024 + (h + 1) * 64]
            sc = _mm_t(q, k, out_dtype=BF16)
            sc = jnp.where(kmask, sc, neg)
            p = _softmax_rows(sc)
            outs.append(_mm(p, v).astype(BF16))
        o_ref[blk] = jnp.concatenate(outs, axis=-1)


def _rt_attn_call(qkv_rt):
    return pl.pallas_call(
        _rt_attn_body,
        grid=(8,),
        in_specs=[pl.BlockSpec((2, 136, 1536), lambda b: (b, 0, 0))],
        out_specs=pl.BlockSpec((2, 136, 512), lambda b: (b, 0, 0)),
        out_shape=jax.ShapeDtypeStruct((16, 136, 512), BF16),
        compiler_params=pltpu.CompilerParams(
            dimension_semantics=("parallel",)),
    )(qkv_rt)


def _ffn_body(attn_ref, xin_ref, ow_ref, nw_ref, up_ref, down_ref, y_ref):
    xo = _mm(attn_ref[...], ow_ref[...]) + xin_ref[...]
    xf = _rms(xo, nw_ref[...])
    u = _mm(xf, up_ref[...])
    x1 = u[:, :2048]
    x2 = u[:, 2048:]
    h = x1 * jax.lax.logistic(x1) * x2
    y_ref[...] = _mm(h, down_ref[...]) + xo


def _ffn_call(attn, xin, ow, nw, up, down, tiles):
    n, d = xin.shape
    tn = n // tiles
    return pl.pallas_call(
        _ffn_body,
        grid=(tiles,),
        in_specs=[
            pl.BlockSpec((tn, d), lambda i: (i, 0)),
            pl.BlockSpec((tn, d), lambda i: (i, 0)),
            pl.BlockSpec((d, d), lambda i: (0, 0)),
            pl.BlockSpec((1, d), lambda i: (0, 0)),
            pl.BlockSpec((d, 4096), lambda i: (0, 0)),
            pl.BlockSpec((2048, d), lambda i: (0, 0)),
        ],
        out_specs=pl.BlockSpec((tn, d), lambda i: (i, 0)),
        out_shape=jax.ShapeDtypeStruct((n, d), F32),
        compiler_params=pltpu.CompilerParams(
            dimension_semantics=("parallel",)),
    )(attn, xin, ow, nw, up, down)


def _final_body(r_ref, ow_ref, krp_ref, kgp_ref, o_ref):
    out64 = _mm(r_ref[...], ow_ref[...])
    for g in range(8):
        t_i = jax.lax.broadcasted_iota(jnp.int32, (16, 64), 0)
        r_i = jax.lax.broadcasted_iota(jnp.int32, (16, 64), 1)
        sel = (r_i == ((t_i + 15) % 16) * 4 + (g // 2)).astype(F32)
        xsel = _mm(sel, out64)
        o_ref[0, g * 16:(g + 1) * 16, :] = _mm(xsel[:, :128], krp_ref[g])
        o_ref[1, g * 16:(g + 1) * 16, :] = _mm(xsel[:, 128:], kgp_ref[g])


def _final_call(r_tok, ow, krp, kgp):
    return pl.pallas_call(
        _final_body,
        in_specs=[
            pl.BlockSpec((64, 512), lambda: (0, 0)),
            pl.BlockSpec((512, 256), lambda: (0, 0)),
            pl.BlockSpec((8, 128, 96), lambda: (0, 0, 0)),
            pl.BlockSpec((8, 128, 96), lambda: (0, 0, 0)),
        ],
        out_specs=pl.BlockSpec((2, 128, 96), lambda: (0, 0, 0)),
        out_shape=jax.ShapeDtypeStruct((2, 128, 96), F32),
    )(r_tok, ow, krp, kgp)


def kernel(x, doc,
           enc0_attn_w, enc0_attn_o_w, enc0_ffn_up_w, enc0_ffn_down_w,
           enc0_attn_norm_w, enc0_ffn_norm_w,
           enc1_attn_w, enc1_attn_o_w, enc1_ffn_up_w, enc1_ffn_down_w,
           enc1_attn_norm_w, enc1_ffn_norm_w,
           rt0_attn_w, rt0_attn_o_w, rt0_ffn_up_w, rt0_ffn_down_w,
           rt0_attn_norm_w, rt0_ffn_norm_w,
           rt1_attn_w, rt1_attn_o_w, rt1_ffn_up_w, rt1_ffn_down_w,
           rt1_attn_norm_w, rt1_ffn_norm_w,
           router_token, out_w, keys_router, keys_gate):
    bf = lambda t: t.astype(BF16)
    x2 = x.reshape(2048, 512)
    doc_r = doc.reshape(1, 2048).astype(jnp.int32)
    doc_c = doc_r.reshape(2048, 1)
    tbl_enc, tbl_rt = _tables_call()

    enc = [(enc0_attn_w, enc0_attn_o_w, enc0_ffn_up_w, enc0_ffn_down_w,
            enc0_attn_norm_w, enc0_ffn_norm_w),
           (enc1_attn_w, enc1_attn_o_w, enc1_ffn_up_w, enc1_ffn_down_w,
            enc1_attn_norm_w, enc1_ffn_norm_w)]
    for aw, ow, up, down, anw, fnw in enc:
        qkv = _qkv_call(x2, anw.reshape(1, 512), bf(aw[:, _QKV_PERM]),
                        tbl_enc, tiles=4)
        attn = _enc_attn_call(doc_c, doc_r, qkv)
        x2 = _ffn_call(attn, x2, bf(ow), fnw.reshape(1, 512), bf(up),
                       bf(down), tiles=8)

    xb = x2.reshape(16, 128, 512)
    rt_tok = jnp.broadcast_to(router_token, (16, 4, 512))
    pad = jnp.zeros((16, 4, 512), F32)
    xflat = jnp.concatenate([xb, rt_tok, pad], axis=1).reshape(2176, 512)

    rt = [(rt0_attn_w, rt0_attn_o_w, rt0_ffn_up_w, rt0_ffn_down_w,
           rt0_attn_norm_w, rt0_ffn_norm_w),
          (rt1_attn_w, rt1_attn_o_w, rt1_ffn_up_w, rt1_ffn_down_w,
           rt1_attn_norm_w, rt1_ffn_norm_w)]
    for aw, ow, up, down, anw, fnw in rt:
        attn = _rt_fused_call(xflat, anw.reshape(1, 512), bf(aw[:, _QKV_PERM]),
                              tbl_rt)
        xflat = _ffn_call(attn, xflat, bf(ow),
                          fnw.reshape(1, 512), bf(up), bf(down), tiles=8)

    r_tok = xflat.reshape(16, 136, 512)[:, 128:132, :].reshape(64, 512)
    krp = keys_router.reshape(12, 8, 128, 8).transpose(1, 2, 0, 3)
    kgp = keys_gate.reshape(12, 8, 128, 8).transpose(1, 2, 0, 3)
    o = _final_call(r_tok, bf(out_w), bf(krp.reshape(8, 128, 96)),
                    bf(kgp.reshape(8, 128, 96)))
    lr = o[0].reshape(8, 16, 12, 8).transpose(2, 0, 1, 3)
    lg = o[1].reshape(8, 16, 12, 8).transpose(2, 0, 1, 3)
    return jnp.concatenate([lr, lg], axis=-1)


# speedup vs baseline: 1.0969x; 1.0969x over previous
import functools
import math

import numpy as np

import jax
import jax.numpy as jnp
from jax.experimental import pallas as pl
from jax.experimental.pallas import tpu as pltpu

F32 = jnp.float32
BF16 = jnp.bfloat16
NEG = -1e30
LN_THETA = math.log(10000.0)
EPS = 1e-5

_h = np.arange(8)[:, None] * 64 + np.arange(32)[None, :]
_qx1 = _h.reshape(256)
_QKV_PERM = np.concatenate([_qx1, _qx1 + 32, _qx1 + 512, _qx1 + 544,
                            np.arange(1024, 1536)])


def _mm(a, b):
    return jax.lax.dot_general(
        a.astype(BF16), b.astype(BF16), (((1,), (0,)), ((), ())),
        preferred_element_type=F32)


def _mm_t(a, b, out_dtype=F32):
    r = jax.lax.dot_general(
        a.astype(BF16), b.astype(BF16), (((1,), (1,)), ((), ())),
        preferred_element_type=F32)
    return r.astype(out_dtype)


def _rms(x, w):
    return x * jax.lax.rsqrt(jnp.mean(x * x, axis=-1, keepdims=True) + EPS) * w


def _softmax_rows(sc):
    e = jnp.exp(sc)
    s = jnp.sum(e, axis=-1, keepdims=True, dtype=F32)
    return e * (1.0 / s).astype(sc.dtype)


def _tables_body(enc_ref, rt_ref):
    j = jax.lax.broadcasted_iota(jnp.int32, (2048, 32), 1).astype(F32)
    inv = jnp.exp(j * (-LN_THETA / 32.0))
    pos = jax.lax.broadcasted_iota(jnp.int32, (2048, 32), 0)
    f = pos.astype(F32) * inv
    enc_ref[:, :32] = jnp.cos(f)
    enc_ref[:, 32:] = jnp.sin(f)
    t136 = jnp.concatenate([jnp.cos(f[:136]), jnp.sin(f[:136])], axis=-1)
    for i in range(16):
        rt_ref[i * 136:(i + 1) * 136, :] = t136


def _tables_call():
    return pl.pallas_call(
        _tables_body,
        out_specs=[pl.BlockSpec((2048, 64), lambda: (0, 0)),
                   pl.BlockSpec((2176, 64), lambda: (0, 0))],
        out_shape=[jax.ShapeDtypeStruct((2048, 64), F32),
                   jax.ShapeDtypeStruct((2176, 64), F32)],
    )()


def _qkv_body(x_ref, nw_ref, w_ref, tbl_ref, o_ref):
    xn = _rms(x_ref[...], nw_ref[...])
    qkv = _mm(xn, w_ref[...])
    c32 = tbl_ref[:, :32]
    s32 = tbl_ref[:, 32:]
    c = jnp.concatenate([c32, c32, c32, c32], axis=-1)
    s = jnp.concatenate([s32, s32, s32, s32], axis=-1)
    c = jnp.concatenate([c, c], axis=-1)
    s = jnp.concatenate([s, s], axis=-1)
    for base, cc, ss in ((0, c * 0.125, s * 0.125), (512, c, s)):
        a = qkv[:, base:base + 256]
        b = qkv[:, base + 256:base + 512]
        o_ref[:, base:base + 256] = (a * cc + b * ss).astype(BF16)
        o_ref[:, base + 256:base + 512] = (b * cc - a * ss).astype(BF16)
    o_ref[:, 1024:] = qkv[:, 1024:].astype(BF16)


def _qkv_call(x, nw, w, tbl, tiles):
    n, d = x.shape
    dout = w.shape[1]
    tn = n // tiles
    return pl.pallas_call(
        _qkv_body,
        grid=(tiles,),
        in_specs=[
            pl.BlockSpec((tn, d), lambda i: (i, 0)),
            pl.BlockSpec((1, d), lambda i: (0, 0)),
            pl.BlockSpec((d, dout), lambda i: (0, 0)),
            pl.BlockSpec((tn, 64), lambda i: (i, 0)),
        ],
        out_specs=pl.BlockSpec((tn, dout), lambda i: (i, 0)),
        out_shape=jax.ShapeDtypeStruct((n, dout), BF16),
        compiler_params=pltpu.CompilerParams(
            dimension_semantics=("parallel",)),
    )(x, nw, w, tbl)


def _enc_attn_body(doc_c_ref, doc_r_ref, qa_ref, qb_ref, ka_ref, kb_ref,
                   v_ref, o_ref, *, qt, kw):
    rb = (qt * 512 + jax.lax.broadcasted_iota(jnp.int32, (512, 1), 0)) // 128
    cb = jax.lax.broadcasted_iota(jnp.int32, (1, kw), 1) // 128
    mask = (rb >= cb) & (doc_c_ref[...] == doc_r_ref[...])
    neg = jnp.asarray(NEG, BF16)
    outs = []
    for h in range(4):
        q = jnp.concatenate([qa_ref[:, h * 32:(h + 1) * 32],
                             qb_ref[:, h * 32:(h + 1) * 32]], axis=-1)
        k = jnp.concatenate([ka_ref[:, h * 32:(h + 1) * 32],
                             kb_ref[:, h * 32:(h + 1) * 32]], axis=-1)
        v = v_ref[:, h * 64:(h + 1) * 64]
        sc = _mm_t(q, k, out_dtype=BF16)
        sc = jnp.where(mask, sc, neg)
        p = _softmax_rows(sc)
        outs.append(_mm(p, v).astype(BF16))
    o_ref[...] = jnp.concatenate(outs, axis=-1)


def _enc_attn_call(doc_c, doc_r, qkv, qt):
    kw = (qt + 1) * 512
    body = functools.partial(_enc_attn_body, qt=qt, kw=kw)
    return pl.pallas_call(
        body,
        grid=(2,),
        in_specs=[
            pl.BlockSpec((512, 1), lambda hp, qt=qt: (qt, 0)),
            pl.BlockSpec((1, kw), lambda hp: (0, 0)),
            pl.BlockSpec((512, 128), lambda hp, qt=qt: (qt, hp)),
            pl.BlockSpec((512, 128), lambda hp, qt=qt: (qt, 2 + hp)),
            pl.BlockSpec((kw, 128), lambda hp: (0, 4 + hp)),
            pl.BlockSpec((kw, 128), lambda hp: (0, 6 + hp)),
            pl.BlockSpec((kw, 256), lambda hp: (0, 4 + hp)),
        ],
        out_specs=pl.BlockSpec((512, 256), lambda hp: (0, hp)),
        out_shape=jax.ShapeDtypeStruct((512, 512), BF16),
        compiler_params=pltpu.CompilerParams(
            dimension_semantics=("parallel",)),
    )(doc_c, doc_r, qkv, qkv, qkv, qkv, qkv)


def _rt_fused_body(x_ref, nw_ref, w_ref, tbl_ref, o_ref):
    xn = _rms(x_ref[...], nw_ref[...])
    qkv = _mm(xn, w_ref[...])
    c32 = tbl_ref[:, :32]
    s32 = tbl_ref[:, 32:]
    c = jnp.concatenate([c32, c32, c32, c32], axis=-1)
    s = jnp.concatenate([s32, s32, s32, s32], axis=-1)
    c = jnp.concatenate([c, c], axis=-1)
    s = jnp.concatenate([s, s], axis=-1)
    rot = []
    for base, cc, ss in ((0, c * 0.125, s * 0.125), (512, c, s)):
        a = qkv[:, base:base + 256]
        b = qkv[:, base + 256:base + 512]
        rot.append((a * cc + b * ss).astype(BF16))
        rot.append((b * cc - a * ss).astype(BF16))
    qa, qb, ka, kb = rot
    v = qkv[:, 1024:].astype(BF16)
    kmask = jax.lax.broadcasted_iota(jnp.int32, (1, 136), 1) < 132
    neg = jnp.asarray(NEG, BF16)
    blocks = []
    for blk in range(4):
        r = slice(blk * 136, (blk + 1) * 136)
        outs = []
        for h in range(8):
            q = jnp.concatenate([qa[r, h * 32:(h + 1) * 32],
                                 qb[r, h * 32:(h + 1) * 32]], axis=-1)
            k = jnp.concatenate([ka[r, h * 32:(h + 1) * 32],
                                 kb[r, h * 32:(h + 1) * 32]], axis=-1)
            sc = _mm_t(q, k, out_dtype=BF16)
            sc = jnp.where(kmask, sc, neg)
            p = _softmax_rows(sc)
            outs.append(_mm(p, v[r, h * 64:(h + 1) * 64]).astype(BF16))
        blocks.append(jnp.concatenate(outs, axis=-1))
    o_ref[...] = jnp.concatenate(blocks, axis=0)


def _rt_fused_call(x, nw, w, tbl):
    return pl.pallas_call(
        _rt_fused_body,
        grid=(4,),
        in_specs=[
            pl.BlockSpec((544, 512), lambda i: (i, 0)),
            pl.BlockSpec((1, 512), lambda i: (0, 0)),
            pl.BlockSpec((512, 1536), lambda i: (0, 0)),
            pl.BlockSpec((544, 64), lambda i: (i, 0)),
        ],
        out_specs=pl.BlockSpec((544, 512), lambda i: (i, 0)),
        out_shape=jax.ShapeDtypeStruct((2176, 512), BF16),
        compiler_params=pltpu.CompilerParams(
            dimension_semantics=("parallel",)),
    )(x, nw, w, tbl)


def _rt_attn_body(z_ref, o_ref):
    kmask = jax.lax.broadcasted_iota(jnp.int32, (1, 136), 1) < 132
    neg = jnp.asarray(NEG, BF16)
    for blk in range(2):
        z = z_ref[blk]
        outs = []
        for h in range(8):
            q = jnp.concatenate([z[:, h * 32:(h + 1) * 32],
                                 z[:, 256 + h * 32:256 + (h + 1) * 32]], axis=-1)
            k = jnp.concatenate([z[:, 512 + h * 32:512 + (h + 1) * 32],
                                 z[:, 768 + h * 32:768 + (h + 1) * 32]], axis=-1)
            v = z[:, 1024 + h * 64:1024 + (h + 1) * 64]
            sc = _mm_t(q, k, out_dtype=BF16)
            sc = jnp.where(kmask, sc, neg)
            p = _softmax_rows(sc)
            outs.append(_mm(p, v).astype(BF16))
        o_ref[blk] = jnp.concatenate(outs, axis=-1)


def _rt_attn_call(qkv_rt):
    return pl.pallas_call(
        _rt_attn_body,
        grid=(8,),
        in_specs=[pl.BlockSpec((2, 136, 1536), lambda b: (b, 0, 0))],
        out_specs=pl.BlockSpec((2, 136, 512), lambda b: (b, 0, 0)),
        out_shape=jax.ShapeDtypeStruct((16, 136, 512), BF16),
        compiler_params=pltpu.CompilerParams(
            dimension_semantics=("parallel",)),
    )(qkv_rt)


def _ffn_body(attn_ref, xin_ref, ow_ref, nw_ref, up_ref, down_ref, y_ref):
    xo = _mm(attn_ref[...], ow_ref[...]) + xin_ref[...]
    xf = _rms(xo, nw_ref[...])
    u = _mm(xf, up_ref[...])
    x1 = u[:, :2048]
    x2 = u[:, 2048:]
    h = x1 * jax.lax.logistic(x1) * x2
    y_ref[...] = _mm(h, down_ref[...]) + xo


def _ffn_call(attn, xin, ow, nw, up, down, tiles):
    n, d = xin.shape
    tn = n // tiles
    return pl.pallas_call(
        _ffn_body,
        grid=(tiles,),
        in_specs=[
            pl.BlockSpec((tn, d), lambda i: (i, 0)),
            pl.BlockSpec((tn, d), lambda i: (i, 0)),
            pl.BlockSpec((d, d), lambda i: (0, 0)),
            pl.BlockSpec((1, d), lambda i: (0, 0)),
            pl.BlockSpec((d, 4096), lambda i: (0, 0)),
            pl.BlockSpec((2048, d), lambda i: (0, 0)),
        ],
        out_specs=pl.BlockSpec((tn, d), lambda i: (i, 0)),
        out_shape=jax.ShapeDtypeStruct((n, d), F32),
        compiler_params=pltpu.CompilerParams(
            dimension_semantics=("parallel",)),
    )(attn, xin, ow, nw, up, down)


def _final_body(r_ref, ow_ref, krp_ref, kgp_ref, o_ref):
    out64 = _mm(r_ref[...], ow_ref[...])
    for g in range(8):
        t_i = jax.lax.broadcasted_iota(jnp.int32, (16, 64), 0)
        r_i = jax.lax.broadcasted_iota(jnp.int32, (16, 64), 1)
        sel = (r_i == ((t_i + 15) % 16) * 4 + (g // 2)).astype(F32)
        xsel = _mm(sel, out64)
        o_ref[0, g * 16:(g + 1) * 16, :] = _mm(xsel[:, :128], krp_ref[g])
        o_ref[1, g * 16:(g + 1) * 16, :] = _mm(xsel[:, 128:], kgp_ref[g])


def _final_call(r_tok, ow, krp, kgp):
    return pl.pallas_call(
        _final_body,
        in_specs=[
            pl.BlockSpec((64, 512), lambda: (0, 0)),
            pl.BlockSpec((512, 256), lambda: (0, 0)),
            pl.BlockSpec((8, 128, 96), lambda: (0, 0, 0)),
            pl.BlockSpec((8, 128, 96), lambda: (0, 0, 0)),
        ],
        out_specs=pl.BlockSpec((2, 128, 96), lambda: (0, 0, 0)),
        out_shape=jax.ShapeDtypeStruct((2, 128, 96), F32),
    )(r_tok, ow, krp, kgp)


def kernel(x, doc,
           enc0_attn_w, enc0_attn_o_w, enc0_ffn_up_w, enc0_ffn_down_w,
           enc0_attn_norm_w, enc0_ffn_norm_w,
           enc1_attn_w, enc1_attn_o_w, enc1_ffn_up_w, enc1_ffn_down_w,
           enc1_attn_norm_w, enc1_ffn_norm_w,
           rt0_attn_w, rt0_attn_o_w, rt0_ffn_up_w, rt0_ffn_down_w,
           rt0_attn_norm_w, rt0_ffn_norm_w,
           rt1_attn_w, rt1_attn_o_w, rt1_ffn_up_w, rt1_ffn_down_w,
           rt1_attn_norm_w, rt1_ffn_norm_w,
           router_token, out_w, keys_router, keys_gate):
    bf = lambda t: t.astype(BF16)
    x2 = x.reshape(2048, 512)
    doc_r = doc.reshape(1, 2048).astype(jnp.int32)
    doc_c = doc_r.reshape(2048, 1)
    tbl_enc, tbl_rt = _tables_call()

    enc = [(enc0_attn_w, enc0_attn_o_w, enc0_ffn_up_w, enc0_ffn_down_w,
            enc0_attn_norm_w, enc0_ffn_norm_w),
           (enc1_attn_w, enc1_attn_o_w, enc1_ffn_up_w, enc1_ffn_down_w,
            enc1_attn_norm_w, enc1_ffn_norm_w)]
    for aw, ow, up, down, anw, fnw in enc:
        qkv = _qkv_call(x2, anw.reshape(1, 512), bf(aw[:, _QKV_PERM]),
                        tbl_enc, tiles=4)
        attn = jnp.concatenate(
            [_enc_attn_call(doc_c, doc_r, qkv, qt) for qt in range(4)], axis=0)
        x2 = _ffn_call(attn, x2, bf(ow), fnw.reshape(1, 512), bf(up),
                       bf(down), tiles=4)

    xb = x2.reshape(16, 128, 512)
    rt_tok = jnp.broadcast_to(router_token, (16, 4, 512))
    pad = jnp.zeros((16, 4, 512), F32)
    xflat = jnp.concatenate([xb, rt_tok, pad], axis=1).reshape(2176, 512)

    rt = [(rt0_attn_w, rt0_attn_o_w, rt0_ffn_up_w, rt0_ffn_down_w,
           rt0_attn_norm_w, rt0_ffn_norm_w),
          (rt1_attn_w, rt1_attn_o_w, rt1_ffn_up_w, rt1_ffn_down_w,
           rt1_attn_norm_w, rt1_ffn_norm_w)]
    for aw, ow, up, down, anw, fnw in rt:
        attn = _rt_fused_call(xflat, anw.reshape(1, 512), bf(aw[:, _QKV_PERM]),
                              tbl_rt)
        xflat = _ffn_call(attn, xflat, bf(ow),
                          fnw.reshape(1, 512), bf(up), bf(down), tiles=4)

    r_tok = xflat.reshape(16, 136, 512)[:, 128:132, :].reshape(64, 512)
    krp = keys_router.reshape(12, 8, 128, 8).transpose(1, 2, 0, 3)
    kgp = keys_gate.reshape(12, 8, 128, 8).transpose(1, 2, 0, 3)
    o = _final_call(r_tok, bf(out_w), bf(krp.reshape(8, 128, 96)),
                    bf(kgp.reshape(8, 128, 96)))
    lr = o[0].reshape(8, 16, 12, 8).transpose(2, 0, 1, 3)
    lg = o[1].reshape(8, 16, 12, 8).transpose(2, 0, 1, 3)
    return jnp.concatenate([lr, lg], axis=-1)


# speedup vs baseline: 1.1163x; 1.0177x over previous
import functools
import math

import numpy as np

import jax
import jax.numpy as jnp
from jax.experimental import pallas as pl
from jax.experimental.pallas import tpu as pltpu

F32 = jnp.float32
BF16 = jnp.bfloat16
NEG = -1e30
LN_THETA = math.log(10000.0)
EPS = 1e-5

_h = np.arange(8)[:, None] * 64 + np.arange(32)[None, :]
_qx1 = _h.reshape(256)
_QKV_PERM = np.concatenate([_qx1, _qx1 + 32, _qx1 + 512, _qx1 + 544,
                            np.arange(1024, 1536)])


def _mm(a, b):
    return jax.lax.dot_general(
        a.astype(BF16), b.astype(BF16), (((1,), (0,)), ((), ())),
        preferred_element_type=F32)


def _mm_t(a, b, out_dtype=F32):
    r = jax.lax.dot_general(
        a.astype(BF16), b.astype(BF16), (((1,), (1,)), ((), ())),
        preferred_element_type=F32)
    return r.astype(out_dtype)


def _rms(x, w):
    return x * jax.lax.rsqrt(jnp.mean(x * x, axis=-1, keepdims=True) + EPS) * w


def _softmax_rows(sc):
    e = jnp.exp(sc)
    s = jnp.sum(e, axis=-1, keepdims=True, dtype=F32)
    return e * (1.0 / s).astype(sc.dtype)


def _tables_body(enc_ref, rt_ref):
    j = jax.lax.broadcasted_iota(jnp.int32, (2048, 32), 1).astype(F32)
    inv = jnp.exp(j * (-LN_THETA / 32.0))
    pos = jax.lax.broadcasted_iota(jnp.int32, (2048, 32), 0)
    f = pos.astype(F32) * inv
    enc_ref[:, :32] = jnp.cos(f)
    enc_ref[:, 32:] = jnp.sin(f)
    t136 = jnp.concatenate([jnp.cos(f[:136]), jnp.sin(f[:136])], axis=-1)
    for i in range(16):
        rt_ref[i * 136:(i + 1) * 136, :] = t136


def _tables_call():
    return pl.pallas_call(
        _tables_body,
        out_specs=[pl.BlockSpec((2048, 64), lambda: (0, 0)),
                   pl.BlockSpec((2176, 64), lambda: (0, 0))],
        out_shape=[jax.ShapeDtypeStruct((2048, 64), F32),
                   jax.ShapeDtypeStruct((2176, 64), F32)],
    )()


def _qkv_body(x_ref, nw_ref, w_ref, tbl_ref, o_ref):
    xn = _rms(x_ref[...], nw_ref[...])
    qkv = _mm(xn, w_ref[...])
    c32 = tbl_ref[:, :32]
    s32 = tbl_ref[:, 32:]
    c = jnp.concatenate([c32, c32, c32, c32], axis=-1)
    s = jnp.concatenate([s32, s32, s32, s32], axis=-1)
    c = jnp.concatenate([c, c], axis=-1)
    s = jnp.concatenate([s, s], axis=-1)
    for base, cc, ss in ((0, c * 0.125, s * 0.125), (512, c, s)):
        a = qkv[:, base:base + 256]
        b = qkv[:, base + 256:base + 512]
        o_ref[:, base:base + 256] = (a * cc + b * ss).astype(BF16)
        o_ref[:, base + 256:base + 512] = (b * cc - a * ss).astype(BF16)
    o_ref[:, 1024:] = qkv[:, 1024:].astype(BF16)


def _qkv_call(x, nw, w, tbl, tiles):
    n, d = x.shape
    dout = w.shape[1]
    tn = n // tiles
    return pl.pallas_call(
        _qkv_body,
        grid=(tiles,),
        in_specs=[
            pl.BlockSpec((tn, d), lambda i: (i, 0)),
            pl.BlockSpec((1, d), lambda i: (0, 0)),
            pl.BlockSpec((d, dout), lambda i: (0, 0)),
            pl.BlockSpec((tn, 64), lambda i: (i, 0)),
        ],
        out_specs=pl.BlockSpec((tn, dout), lambda i: (i, 0)),
        out_shape=jax.ShapeDtypeStruct((n, dout), BF16),
        compiler_params=pltpu.CompilerParams(
            dimension_semantics=("parallel",)),
    )(x, nw, w, tbl)


def _enc_attn_body(doc_c_ref, doc_r_ref, qa_ref, qb_ref, ka_ref, kb_ref,
                   v_ref, o_ref):
    step = pl.program_id(0)
    neg = jnp.asarray(NEG, BF16)
    for s in range(4):
        @pl.when(step == s)
        def _(s=s):
            qt = (3 * s) % 4
            kw = (qt + 1) * 512
            rb = (qt * 512
                  + jax.lax.broadcasted_iota(jnp.int32, (512, 1), 0)) // 128
            cb = jax.lax.broadcasted_iota(jnp.int32, (1, kw), 1) // 128
            mask = (rb >= cb) & (doc_c_ref[...] == doc_r_ref[0:1, 0:kw])
            outs = []
            for h in range(4):
                q = jnp.concatenate([qa_ref[:, h * 32:(h + 1) * 32],
                                     qb_ref[:, h * 32:(h + 1) * 32]], axis=-1)
                k = jnp.concatenate([ka_ref[0:kw, h * 32:(h + 1) * 32],
                                     kb_ref[0:kw, h * 32:(h + 1) * 32]],
                                    axis=-1)
                v = v_ref[0:kw, h * 64:(h + 1) * 64]
                sc = _mm_t(q, k, out_dtype=BF16)
                sc = jnp.where(mask, sc, neg)
                p = _softmax_rows(sc)
                outs.append(_mm(p, v).astype(BF16))
            o_ref[...] = jnp.concatenate(outs, axis=-1)


def _enc_attn_call(doc_c, doc_r, qkv):
    perm = lambda s: (3 * s) % 4
    return pl.pallas_call(
        _enc_attn_body,
        grid=(4, 2),
        in_specs=[
            pl.BlockSpec((512, 1), lambda s, hp: (perm(s), 0)),
            pl.BlockSpec((1, 2048), lambda s, hp: (0, 0)),
            pl.BlockSpec((512, 128), lambda s, hp: (perm(s), hp)),
            pl.BlockSpec((512, 128), lambda s, hp: (perm(s), 2 + hp)),
            pl.BlockSpec((2048, 128), lambda s, hp: (0, 4 + hp)),
            pl.BlockSpec((2048, 128), lambda s, hp: (0, 6 + hp)),
            pl.BlockSpec((2048, 256), lambda s, hp: (0, 4 + hp)),
        ],
        out_specs=pl.BlockSpec((512, 256), lambda s, hp: (perm(s), hp)),
        out_shape=jax.ShapeDtypeStruct((2048, 512), BF16),
        compiler_params=pltpu.CompilerParams(
            dimension_semantics=("parallel", "parallel")),
    )(doc_c, doc_r, qkv, qkv, qkv, qkv, qkv)


def _rt_fused_body(x_ref, nw_ref, w_ref, tbl_ref, o_ref):
    xn = _rms(x_ref[...], nw_ref[...])
    qkv = _mm(xn, w_ref[...])
    c32 = tbl_ref[:, :32]
    s32 = tbl_ref[:, 32:]
    c = jnp.concatenate([c32, c32, c32, c32], axis=-1)
    s = jnp.concatenate([s32, s32, s32, s32], axis=-1)
    c = jnp.concatenate([c, c], axis=-1)
    s = jnp.concatenate([s, s], axis=-1)
    rot = []
    for base, cc, ss in ((0, c * 0.125, s * 0.125), (512, c, s)):
        a = qkv[:, base:base + 256]
        b = qkv[:, base + 256:base + 512]
        rot.append((a * cc + b * ss).astype(BF16))
        rot.append((b * cc - a * ss).astype(BF16))
    qa, qb, ka, kb = rot
    v = qkv[:, 1024:].astype(BF16)
    kmask = jax.lax.broadcasted_iota(jnp.int32, (1, 136), 1) < 132
    neg = jnp.asarray(NEG, BF16)
    blocks = []
    for blk in range(4):
        r = slice(blk * 136, (blk + 1) * 136)
        outs = []
        for h in range(8):
            q = jnp.concatenate([qa[r, h * 32:(h + 1) * 32],
                                 qb[r, h * 32:(h + 1) * 32]], axis=-1)
            k = jnp.concatenate([ka[r, h * 32:(h + 1) * 32],
                                 kb[r, h * 32:(h + 1) * 32]], axis=-1)
            sc = _mm_t(q, k, out_dtype=BF16)
            sc = jnp.where(kmask, sc, neg)
            p = _softmax_rows(sc)
            outs.append(_mm(p, v[r, h * 64:(h + 1) * 64]).astype(BF16))
        blocks.append(jnp.concatenate(outs, axis=-1))
    o_ref[...] = jnp.concatenate(blocks, axis=0)


def _rt_fused_call(x, nw, w, tbl):
    return pl.pallas_call(
        _rt_fused_body,
        grid=(4,),
        in_specs=[
            pl.BlockSpec((544, 512), lambda i: (i, 0)),
            pl.BlockSpec((1, 512), lambda i: (0, 0)),
            pl.BlockSpec((512, 1536), lambda i: (0, 0)),
            pl.BlockSpec((544, 64), lambda i: (i, 0)),
        ],
        out_specs=pl.BlockSpec((544, 512), lambda i: (i, 0)),
        out_shape=jax.ShapeDtypeStruct((2176, 512), BF16),
        compiler_params=pltpu.CompilerParams(
            dimension_semantics=("parallel",)),
    )(x, nw, w, tbl)


def _rt_attn_body(z_ref, o_ref):
    kmask = jax.lax.broadcasted_iota(jnp.int32, (1, 136), 1) < 132
    neg = jnp.asarray(NEG, BF16)
    for blk in range(2):
        z = z_ref[blk]
        outs = []
        for h in range(8):
            q = jnp.concatenate([z[:, h * 32:(h + 1) * 32],
                                 z[:, 256 + h * 32:256 + (h + 1) * 32]], axis=-1)
            k = jnp.concatenate([z[:, 512 + h * 32:512 + (h + 1) * 32],
                                 z[:, 768 + h * 32:768 + (h + 1) * 32]], axis=-1)
            v = z[:, 1024 + h * 64:1024 + (h + 1) * 64]
            sc = _mm_t(q, k, out_dtype=BF16)
            sc = jnp.where(kmask, sc, neg)
            p = _softmax_rows(sc)
            outs.append(_mm(p, v).astype(BF16))
        o_ref[blk] = jnp.concatenate(outs, axis=-1)


def _rt_attn_call(qkv_rt):
    return pl.pallas_call(
        _rt_attn_body,
        grid=(8,),
        in_specs=[pl.BlockSpec((2, 136, 1536), lambda b: (b, 0, 0))],
        out_specs=pl.BlockSpec((2, 136, 512), lambda b: (b, 0, 0)),
        out_shape=jax.ShapeDtypeStruct((16, 136, 512), BF16),
        compiler_params=pltpu.CompilerParams(
            dimension_semantics=("parallel",)),
    )(qkv_rt)


def _ffn_body(attn_ref, xin_ref, ow_ref, nw_ref, up_ref, down_ref, y_ref):
    xo = _mm(attn_ref[...], ow_ref[...]) + xin_ref[...]
    xf = _rms(xo, nw_ref[...])
    u = _mm(xf, up_ref[...])
    x1 = u[:, :2048]
    x2 = u[:, 2048:]
    h = x1 * jax.lax.logistic(x1) * x2
    y_ref[...] = _mm(h, down_ref[...]) + xo


def _ffn_call(attn, xin, ow, nw, up, down, tiles):
    n, d = xin.shape
    tn = n // tiles
    return pl.pallas_call(
        _ffn_body,
        grid=(tiles,),
        in_specs=[
            pl.BlockSpec((tn, d), lambda i: (i, 0)),
            pl.BlockSpec((tn, d), lambda i: (i, 0)),
            pl.BlockSpec((d, d), lambda i: (0, 0)),
            pl.BlockSpec((1, d), lambda i: (0, 0)),
            pl.BlockSpec((d, 4096), lambda i: (0, 0)),
            pl.BlockSpec((2048, d), lambda i: (0, 0)),
        ],
        out_specs=pl.BlockSpec((tn, d), lambda i: (i, 0)),
        out_shape=jax.ShapeDtypeStruct((n, d), F32),
        compiler_params=pltpu.CompilerParams(
            dimension_semantics=("parallel",)),
    )(attn, xin, ow, nw, up, down)


def _final_body(r_ref, ow_ref, krp_ref, kgp_ref, o_ref):
    out64 = _mm(r_ref[...], ow_ref[...])
    for g in range(8):
        t_i = jax.lax.broadcasted_iota(jnp.int32, (16, 64), 0)
        r_i = jax.lax.broadcasted_iota(jnp.int32, (16, 64), 1)
        sel = (r_i == ((t_i + 15) % 16) * 4 + (g // 2)).astype(F32)
        xsel = _mm(sel, out64)
        o_ref[0, g * 16:(g + 1) * 16, :] = _mm(xsel[:, :128], krp_ref[g])
        o_ref[1, g * 16:(g + 1) * 16, :] = _mm(xsel[:, 128:], kgp_ref[g])


def _final_call(r_tok, ow, krp, kgp):
    return pl.pallas_call(
        _final_body,
        in_specs=[
            pl.BlockSpec((64, 512), lambda: (0, 0)),
            pl.BlockSpec((512, 256), lambda: (0, 0)),
            pl.BlockSpec((8, 128, 96), lambda: (0, 0, 0)),
            pl.BlockSpec((8, 128, 96), lambda: (0, 0, 0)),
        ],
        out_specs=pl.BlockSpec((2, 128, 96), lambda: (0, 0, 0)),
        out_shape=jax.ShapeDtypeStruct((2, 128, 96), F32),
    )(r_tok, ow, krp, kgp)


def kernel(x, doc,
           enc0_attn_w, enc0_attn_o_w, enc0_ffn_up_w, enc0_ffn_down_w,
           enc0_attn_norm_w, enc0_ffn_norm_w,
           enc1_attn_w, enc1_attn_o_w, enc1_ffn_up_w, enc1_ffn_down_w,
           enc1_attn_norm_w, enc1_ffn_norm_w,
           rt0_attn_w, rt0_attn_o_w, rt0_ffn_up_w, rt0_ffn_down_w,
           rt0_attn_norm_w, rt0_ffn_norm_w,
           rt1_attn_w, rt1_attn_o_w, rt1_ffn_up_w, rt1_ffn_down_w,
           rt1_attn_norm_w, rt1_ffn_norm_w,
           router_token, out_w, keys_router, keys_gate):
    bf = lambda t: t.astype(BF16)
    x2 = x.reshape(2048, 512)
    doc_r = doc.reshape(1, 2048).astype(jnp.int32)
    doc_c = doc_r.reshape(2048, 1)
    tbl_enc, tbl_rt = _tables_call()

    enc = [(enc0_attn_w, enc0_attn_o_w, enc0_ffn_up_w, enc0_ffn_down_w,
            enc0_attn_norm_w, enc0_ffn_norm_w),
           (enc1_attn_w, enc1_attn_o_w, enc1_ffn_up_w, enc1_ffn_down_w,
            enc1_attn_norm_w, enc1_ffn_norm_w)]
    for aw, ow, up, down, anw, fnw in enc:
        qkv = _qkv_call(x2, anw.reshape(1, 512), bf(aw[:, _QKV_PERM]),
                        tbl_enc, tiles=4)
        attn = _enc_attn_call(doc_c, doc_r, qkv)
        x2 = _ffn_call(attn, x2, bf(ow), fnw.reshape(1, 512), bf(up),
                       bf(down), tiles=4)

    xb = x2.reshape(16, 128, 512)
    rt_tok = jnp.broadcast_to(router_token, (16, 4, 512))
    pad = jnp.zeros((16, 4, 512), F32)
    xflat = jnp.concatenate([xb, rt_tok, pad], axis=1).reshape(2176, 512)

    rt = [(rt0_attn_w, rt0_attn_o_w, rt0_ffn_up_w, rt0_ffn_down_w,
           rt0_attn_norm_w, rt0_ffn_norm_w),
          (rt1_attn_w, rt1_attn_o_w, rt1_ffn_up_w, rt1_ffn_down_w,
           rt1_attn_norm_w, rt1_ffn_norm_w)]
    for aw, ow, up, down, anw, fnw in rt:
        attn = _rt_fused_call(xflat, anw.reshape(1, 512), bf(aw[:, _QKV_PERM]),
                              tbl_rt)
        xflat = _ffn_call(attn, xflat, bf(ow),
                          fnw.reshape(1, 512), bf(up), bf(down), tiles=4)

    r_tok = xflat.reshape(16, 136, 512)[:, 128:132, :].reshape(64, 512)
    krp = keys_router.reshape(12, 8, 128, 8).transpose(1, 2, 0, 3)
    kgp = keys_gate.reshape(12, 8, 128, 8).transpose(1, 2, 0, 3)
    o = _final_call(r_tok, bf(out_w), bf(krp.reshape(8, 128, 96)),
                    bf(kgp.reshape(8, 128, 96)))
    lr = o[0].reshape(8, 16, 12, 8).transpose(2, 0, 1, 3)
    lg = o[1].reshape(8, 16, 12, 8).transpose(2, 0, 1, 3)
    return jnp.concatenate([lr, lg], axis=-1)


# speedup vs baseline: 1.1620x; 1.0409x over previous
import functools
import math

import numpy as np

import jax
import jax.numpy as jnp
from jax.experimental import pallas as pl
from jax.experimental.pallas import tpu as pltpu

F32 = jnp.float32
BF16 = jnp.bfloat16
NEG = -1e30
LN_THETA = math.log(10000.0)
EPS = 1e-5

_h = np.arange(8)[:, None] * 64 + np.arange(32)[None, :]
_qx1 = _h.reshape(256)
_QKV_PERM = np.concatenate([_qx1, _qx1 + 32, _qx1 + 512, _qx1 + 544,
                            np.arange(1024, 1536)])


def _mm(a, b):
    return jax.lax.dot_general(
        a.astype(BF16), b.astype(BF16), (((1,), (0,)), ((), ())),
        preferred_element_type=F32)


def _mm_t(a, b, out_dtype=F32):
    r = jax.lax.dot_general(
        a.astype(BF16), b.astype(BF16), (((1,), (1,)), ((), ())),
        preferred_element_type=F32)
    return r.astype(out_dtype)


def _rms(x, w):
    return x * jax.lax.rsqrt(jnp.mean(x * x, axis=-1, keepdims=True) + EPS) * w


def _softmax_rows(sc):
    e = jnp.exp(sc)
    s = jnp.sum(e, axis=-1, keepdims=True, dtype=F32)
    return e * (1.0 / s).astype(sc.dtype)


def _tables_body(enc_ref, rt_ref):
    j = jax.lax.broadcasted_iota(jnp.int32, (2048, 32), 1).astype(F32)
    inv = jnp.exp(j * (-LN_THETA / 32.0))
    pos = jax.lax.broadcasted_iota(jnp.int32, (2048, 32), 0)
    f = pos.astype(F32) * inv
    enc_ref[:, :32] = jnp.cos(f)
    enc_ref[:, 32:] = jnp.sin(f)
    t136 = jnp.concatenate([jnp.cos(f[:136]), jnp.sin(f[:136])], axis=-1)
    for i in range(16):
        rt_ref[i * 136:(i + 1) * 136, :] = t136


def _tables_call():
    return pl.pallas_call(
        _tables_body,
        out_specs=[pl.BlockSpec((2048, 64), lambda: (0, 0)),
                   pl.BlockSpec((2176, 64), lambda: (0, 0))],
        out_shape=[jax.ShapeDtypeStruct((2048, 64), F32),
                   jax.ShapeDtypeStruct((2176, 64), F32)],
    )()


def _rot_store(qkv, tbl_ref, o_ref):
    c32 = tbl_ref[:, :32]
    s32 = tbl_ref[:, 32:]
    c = jnp.concatenate([c32, c32, c32, c32], axis=-1)
    s = jnp.concatenate([s32, s32, s32, s32], axis=-1)
    c = jnp.concatenate([c, c], axis=-1)
    s = jnp.concatenate([s, s], axis=-1)
    for base, cc, ss in ((0, c * 0.125, s * 0.125), (512, c, s)):
        a = qkv[:, base:base + 256]
        b = qkv[:, base + 256:base + 512]
        o_ref[:, base:base + 256] = (a * cc + b * ss).astype(BF16)
        o_ref[:, base + 256:base + 512] = (b * cc - a * ss).astype(BF16)
    o_ref[:, 1024:] = qkv[:, 1024:].astype(BF16)


def _qkv_body(x_ref, nw_ref, w_ref, tbl_ref, o_ref):
    xn = _rms(x_ref[...], nw_ref[...])
    _rot_store(_mm(xn, w_ref[...]), tbl_ref, o_ref)


def _qkv_call(x, nw, w, tbl, tiles):
    n, d = x.shape
    dout = w.shape[1]
    tn = n // tiles
    return pl.pallas_call(
        _qkv_body,
        grid=(tiles,),
        in_specs=[
            pl.BlockSpec((tn, d), lambda i: (i, 0)),
            pl.BlockSpec((1, d), lambda i: (0, 0)),
            pl.BlockSpec((d, dout), lambda i: (0, 0)),
            pl.BlockSpec((tn, 64), lambda i: (i, 0)),
        ],
        out_specs=pl.BlockSpec((tn, dout), lambda i: (i, 0)),
        out_shape=jax.ShapeDtypeStruct((n, dout), BF16),
        compiler_params=pltpu.CompilerParams(
            dimension_semantics=("parallel",)),
    )(x, nw, w, tbl)


def _enc_attn_body(doc_c_ref, doc_r_ref, qa_ref, qb_ref, ka_ref, kb_ref,
                   v_ref, o_ref):
    step = pl.program_id(0)
    neg = jnp.asarray(NEG, BF16)
    for s in range(4):
        @pl.when(step == s)
        def _(s=s):
            qt = (3 * s) % 4
            kw = (qt + 1) * 512
            rb = (qt * 512
                  + jax.lax.broadcasted_iota(jnp.int32, (512, 1), 0)) // 128
            cb = jax.lax.broadcasted_iota(jnp.int32, (1, kw), 1) // 128
            mask = (rb >= cb) & (doc_c_ref[...] == doc_r_ref[0:1, 0:kw])
            outs = []
            for h in range(4):
                q = jnp.concatenate([qa_ref[:, h * 32:(h + 1) * 32],
                                     qb_ref[:, h * 32:(h + 1) * 32]], axis=-1)
                k = jnp.concatenate([ka_ref[0:kw, h * 32:(h + 1) * 32],
                                     kb_ref[0:kw, h * 32:(h + 1) * 32]],
                                    axis=-1)
                v = v_ref[0:kw, h * 64:(h + 1) * 64]
                sc = _mm_t(q, k, out_dtype=BF16)
                sc = jnp.where(mask, sc, neg)
                p = _softmax_rows(sc)
                outs.append(_mm(p, v).astype(BF16))
            o_ref[...] = jnp.concatenate(outs, axis=-1)


def _enc_attn_call(doc_c, doc_r, qkv):
    perm = lambda s: (3 * s) % 4
    return pl.pallas_call(
        _enc_attn_body,
        grid=(4, 2),
        in_specs=[
            pl.BlockSpec((512, 1), lambda s, hp: (perm(s), 0)),
            pl.BlockSpec((1, 2048), lambda s, hp: (0, 0)),
            pl.BlockSpec((512, 128), lambda s, hp: (perm(s), hp)),
            pl.BlockSpec((512, 128), lambda s, hp: (perm(s), 2 + hp)),
            pl.BlockSpec((2048, 128), lambda s, hp: (0, 4 + hp)),
            pl.BlockSpec((2048, 128), lambda s, hp: (0, 6 + hp)),
            pl.BlockSpec((2048, 256), lambda s, hp: (0, 4 + hp)),
        ],
        out_specs=pl.BlockSpec((512, 256), lambda s, hp: (perm(s), hp)),
        out_shape=jax.ShapeDtypeStruct((2048, 512), BF16),
        compiler_params=pltpu.CompilerParams(
            dimension_semantics=("parallel", "parallel")),
    )(doc_c, doc_r, qkv, qkv, qkv, qkv, qkv)


def _rt_attn_from_qkv(qkv, tbl_ref):
    c32 = tbl_ref[:, :32]
    s32 = tbl_ref[:, 32:]
    c = jnp.concatenate([c32, c32, c32, c32], axis=-1)
    s = jnp.concatenate([s32, s32, s32, s32], axis=-1)
    c = jnp.concatenate([c, c], axis=-1)
    s = jnp.concatenate([s, s], axis=-1)
    rot = []
    for base, cc, ss in ((0, c * 0.125, s * 0.125), (512, c, s)):
        a = qkv[:, base:base + 256]
        b = qkv[:, base + 256:base + 512]
        rot.append((a * cc + b * ss).astype(BF16))
        rot.append((b * cc - a * ss).astype(BF16))
    qa, qb, ka, kb = rot
    v = qkv[:, 1024:].astype(BF16)
    kmask = jax.lax.broadcasted_iota(jnp.int32, (1, 136), 1) < 132
    neg = jnp.asarray(NEG, BF16)
    blocks = []
    for blk in range(4):
        r = slice(blk * 136, (blk + 1) * 136)
        outs = []
        for h in range(8):
            q = jnp.concatenate([qa[r, h * 32:(h + 1) * 32],
                                 qb[r, h * 32:(h + 1) * 32]], axis=-1)
            k = jnp.concatenate([ka[r, h * 32:(h + 1) * 32],
                                 kb[r, h * 32:(h + 1) * 32]], axis=-1)
            sc = _mm_t(q, k, out_dtype=BF16)
            sc = jnp.where(kmask, sc, neg)
            p = _softmax_rows(sc)
            outs.append(_mm(p, v[r, h * 64:(h + 1) * 64]).astype(BF16))
        blocks.append(jnp.concatenate(outs, axis=-1))
    return jnp.concatenate(blocks, axis=0)


def _rt_fused_body(x_ref, nw_ref, w_ref, tbl_ref, o_ref):
    xn = _rms(x_ref[...], nw_ref[...])
    o_ref[...] = _rt_attn_from_qkv(_mm(xn, w_ref[...]), tbl_ref)


def _rt_fused_call(x, nw, w, tbl):
    return pl.pallas_call(
        _rt_fused_body,
        grid=(4,),
        in_specs=[
            pl.BlockSpec((544, 512), lambda i: (i, 0)),
            pl.BlockSpec((1, 512), lambda i: (0, 0)),
            pl.BlockSpec((512, 1536), lambda i: (0, 0)),
            pl.BlockSpec((544, 64), lambda i: (i, 0)),
        ],
        out_specs=pl.BlockSpec((544, 512), lambda i: (i, 0)),
        out_shape=jax.ShapeDtypeStruct((2176, 512), BF16),
        compiler_params=pltpu.CompilerParams(
            dimension_semantics=("parallel",)),
    )(x, nw, w, tbl)


def _rt_attn_body(z_ref, o_ref):
    kmask = jax.lax.broadcasted_iota(jnp.int32, (1, 136), 1) < 132
    neg = jnp.asarray(NEG, BF16)
    for blk in range(2):
        z = z_ref[blk]
        outs = []
        for h in range(8):
            q = jnp.concatenate([z[:, h * 32:(h + 1) * 32],
                                 z[:, 256 + h * 32:256 + (h + 1) * 32]], axis=-1)
            k = jnp.concatenate([z[:, 512 + h * 32:512 + (h + 1) * 32],
                                 z[:, 768 + h * 32:768 + (h + 1) * 32]], axis=-1)
            v = z[:, 1024 + h * 64:1024 + (h + 1) * 64]
            sc = _mm_t(q, k, out_dtype=BF16)
            sc = jnp.where(kmask, sc, neg)
            p = _softmax_rows(sc)
            outs.append(_mm(p, v).astype(BF16))
        o_ref[blk] = jnp.concatenate(outs, axis=-1)


def _rt_attn_call(qkv_rt):
    return pl.pallas_call(
        _rt_attn_body,
        grid=(8,),
        in_specs=[pl.BlockSpec((2, 136, 1536), lambda b: (b, 0, 0))],
        out_specs=pl.BlockSpec((2, 136, 512), lambda b: (b, 0, 0)),
        out_shape=jax.ShapeDtypeStruct((16, 136, 512), BF16),
        compiler_params=pltpu.CompilerParams(
            dimension_semantics=("parallel",)),
    )(qkv_rt)


def _ffn_body(attn_ref, xin_ref, ow_ref, nw_ref, up_ref, down_ref, y_ref):
    xo = _mm(attn_ref[...], ow_ref[...]) + xin_ref[...]
    xf = _rms(xo, nw_ref[...])
    u = _mm(xf, up_ref[...])
    x1 = u[:, :2048]
    x2 = u[:, 2048:]
    h = x1 * jax.lax.logistic(x1) * x2
    y_ref[...] = _mm(h, down_ref[...]) + xo


def _ffn_call(attn, xin, ow, nw, up, down, tiles):
    n, d = xin.shape
    tn = n // tiles
    return pl.pallas_call(
        _ffn_body,
        grid=(tiles,),
        in_specs=[
            pl.BlockSpec((tn, d), lambda i: (i, 0)),
            pl.BlockSpec((tn, d), lambda i: (i, 0)),
            pl.BlockSpec((d, d), lambda i: (0, 0)),
            pl.BlockSpec((1, d), lambda i: (0, 0)),
            pl.BlockSpec((d, 4096), lambda i: (0, 0)),
            pl.BlockSpec((2048, d), lambda i: (0, 0)),
        ],
        out_specs=pl.BlockSpec((tn, d), lambda i: (i, 0)),
        out_shape=jax.ShapeDtypeStruct((n, d), F32),
        compiler_params=pltpu.CompilerParams(
            dimension_semantics=("parallel",)),
    )(attn, xin, ow, nw, up, down)


def _ffn_qkv_body(attn_ref, xin_ref, ow_ref, nw_ref, up_ref, down_ref,
                  nw2_ref, w2_ref, tbl_ref, y_ref, qkv_ref):
    xo = _mm(attn_ref[...], ow_ref[...]) + xin_ref[...]
    xf = _rms(xo, nw_ref[...])
    u = _mm(xf, up_ref[...])
    x1 = u[:, :2048]
    x2 = u[:, 2048:]
    h = x1 * jax.lax.logistic(x1) * x2
    y = _mm(h, down_ref[...]) + xo
    y_ref[...] = y
    _rot_store(_mm(_rms(y, nw2_ref[...]), w2_ref[...]), tbl_ref, qkv_ref)


def _ffn_qkv_call(attn, xin, ow, nw, up, down, nw2, w2, tbl):
    n, d = xin.shape
    tn = n // 4
    return pl.pallas_call(
        _ffn_qkv_body,
        grid=(4,),
        in_specs=[
            pl.BlockSpec((tn, d), lambda i: (i, 0)),
            pl.BlockSpec((tn, d), lambda i: (i, 0)),
            pl.BlockSpec((d, d), lambda i: (0, 0)),
            pl.BlockSpec((1, d), lambda i: (0, 0)),
            pl.BlockSpec((d, 4096), lambda i: (0, 0)),
            pl.BlockSpec((2048, d), lambda i: (0, 0)),
            pl.BlockSpec((1, d), lambda i: (0, 0)),
            pl.BlockSpec((d, 1536), lambda i: (0, 0)),
            pl.BlockSpec((tn, 64), lambda i: (i, 0)),
        ],
        out_specs=[pl.BlockSpec((tn, d), lambda i: (i, 0)),
                   pl.BlockSpec((tn, 1536), lambda i: (i, 0))],
        out_shape=[jax.ShapeDtypeStruct((n, d), F32),
                   jax.ShapeDtypeStruct((n, 1536), BF16)],
        compiler_params=pltpu.CompilerParams(
            dimension_semantics=("parallel",)),
    )(attn, xin, ow, nw, up, down, nw2, w2, tbl)


def _ffn_rt_attn_body(attn_ref, xin_ref, ow_ref, nw_ref, up_ref, down_ref,
                      nw2_ref, w2_ref, tbl_ref, y_ref, a2_ref):
    xo = _mm(attn_ref[...], ow_ref[...]) + xin_ref[...]
    xf = _rms(xo, nw_ref[...])
    u = _mm(xf, up_ref[...])
    x1 = u[:, :2048]
    x2 = u[:, 2048:]
    h = x1 * jax.lax.logistic(x1) * x2
    y = _mm(h, down_ref[...]) + xo
    y_ref[...] = y
    qkv = _mm(_rms(y, nw2_ref[...]), w2_ref[...])
    a2_ref[...] = _rt_attn_from_qkv(qkv, tbl_ref)


def _ffn_rt_attn_call(attn, xin, ow, nw, up, down, nw2, w2, tbl):
    n, d = xin.shape
    tn = n // 4
    return pl.pallas_call(
        _ffn_rt_attn_body,
        grid=(4,),
        in_specs=[
            pl.BlockSpec((tn, d), lambda i: (i, 0)),
            pl.BlockSpec((tn, d), lambda i: (i, 0)),
            pl.BlockSpec((d, d), lambda i: (0, 0)),
            pl.BlockSpec((1, d), lambda i: (0, 0)),
            pl.BlockSpec((d, 4096), lambda i: (0, 0)),
            pl.BlockSpec((2048, d), lambda i: (0, 0)),
            pl.BlockSpec((1, d), lambda i: (0, 0)),
            pl.BlockSpec((d, 1536), lambda i: (0, 0)),
            pl.BlockSpec((tn, 64), lambda i: (i, 0)),
        ],
        out_specs=[pl.BlockSpec((tn, d), lambda i: (i, 0)),
                   pl.BlockSpec((tn, d), lambda i: (i, 0))],
        out_shape=[jax.ShapeDtypeStruct((n, d), F32),
                   jax.ShapeDtypeStruct((n, d), BF16)],
        compiler_params=pltpu.CompilerParams(
            dimension_semantics=("parallel",)),
    )(attn, xin, ow, nw, up, down, nw2, w2, tbl)


def _final_body(r_ref, ow_ref, krp_ref, kgp_ref, o_ref):
    out64 = _mm(r_ref[...], ow_ref[...])
    for g in range(8):
        t_i = jax.lax.broadcasted_iota(jnp.int32, (16, 64), 0)
        r_i = jax.lax.broadcasted_iota(jnp.int32, (16, 64), 1)
        sel = (r_i == ((t_i + 15) % 16) * 4 + (g // 2)).astype(F32)
        xsel = _mm(sel, out64)
        o_ref[0, g * 16:(g + 1) * 16, :] = _mm(xsel[:, :128], krp_ref[g])
        o_ref[1, g * 16:(g + 1) * 16, :] = _mm(xsel[:, 128:], kgp_ref[g])


def _final_call(r_tok, ow, krp, kgp):
    return pl.pallas_call(
        _final_body,
        in_specs=[
            pl.BlockSpec((64, 512), lambda: (0, 0)),
            pl.BlockSpec((512, 256), lambda: (0, 0)),
            pl.BlockSpec((8, 128, 96), lambda: (0, 0, 0)),
            pl.BlockSpec((8, 128, 96), lambda: (0, 0, 0)),
        ],
        out_specs=pl.BlockSpec((2, 128, 96), lambda: (0, 0, 0)),
        out_shape=jax.ShapeDtypeStruct((2, 128, 96), F32),
    )(r_tok, ow, krp, kgp)


def kernel(x, doc,
           enc0_attn_w, enc0_attn_o_w, enc0_ffn_up_w, enc0_ffn_down_w,
           enc0_attn_norm_w, enc0_ffn_norm_w,
           enc1_attn_w, enc1_attn_o_w, enc1_ffn_up_w, enc1_ffn_down_w,
           enc1_attn_norm_w, enc1_ffn_norm_w,
           rt0_attn_w, rt0_attn_o_w, rt0_ffn_up_w, rt0_ffn_down_w,
           rt0_attn_norm_w, rt0_ffn_norm_w,
           rt1_attn_w, rt1_attn_o_w, rt1_ffn_up_w, rt1_ffn_down_w,
           rt1_attn_norm_w, rt1_ffn_norm_w,
           router_token, out_w, keys_router, keys_gate):
    bf = lambda t: t.astype(BF16)
    x2 = x.reshape(2048, 512)
    doc_r = doc.reshape(1, 2048).astype(jnp.int32)
    doc_c = doc_r.reshape(2048, 1)
    tbl_enc, tbl_rt = _tables_call()

    qkv = _qkv_call(x2, enc0_attn_norm_w.reshape(1, 512),
                    bf(enc0_attn_w[:, _QKV_PERM]), tbl_enc, tiles=4)
    attn = _enc_attn_call(doc_c, doc_r, qkv)
    x2, qkv = _ffn_qkv_call(attn, x2, bf(enc0_attn_o_w),
                            enc0_ffn_norm_w.reshape(1, 512),
                            bf(enc0_ffn_up_w), bf(enc0_ffn_down_w),
                            enc1_attn_norm_w.reshape(1, 512),
                            bf(enc1_attn_w[:, _QKV_PERM]), tbl_enc)
    attn = _enc_attn_call(doc_c, doc_r, qkv)
    x2 = _ffn_call(attn, x2, bf(enc1_attn_o_w),
                   enc1_ffn_norm_w.reshape(1, 512), bf(enc1_ffn_up_w),
                   bf(enc1_ffn_down_w), tiles=4)

    xb = x2.reshape(16, 128, 512)
    rt_tok = jnp.broadcast_to(router_token, (16, 4, 512))
    pad = jnp.zeros((16, 4, 512), F32)
    xflat = jnp.concatenate([xb, rt_tok, pad], axis=1).reshape(2176, 512)

    attn = _rt_fused_call(xflat, rt0_attn_norm_w.reshape(1, 512),
                          bf(rt0_attn_w[:, _QKV_PERM]), tbl_rt)
    xflat, attn = _ffn_rt_attn_call(attn, xflat, bf(rt0_attn_o_w),
                                    rt0_ffn_norm_w.reshape(1, 512),
                                    bf(rt0_ffn_up_w), bf(rt0_ffn_down_w),
                                    rt1_attn_norm_w.reshape(1, 512),
                                    bf(rt1_attn_w[:, _QKV_PERM]), tbl_rt)
    xflat = _ffn_call(attn, xflat, bf(rt1_attn_o_w),
                      rt1_ffn_norm_w.reshape(1, 512), bf(rt1_ffn_up_w),
                      bf(rt1_ffn_down_w), tiles=4)

    r_tok = xflat.reshape(16, 136, 512)[:, 128:132, :].reshape(64, 512)
    krp = keys_router.reshape(12, 8, 128, 8).transpose(1, 2, 0, 3)
    kgp = keys_gate.reshape(12, 8, 128, 8).transpose(1, 2, 0, 3)
    o = _final_call(r_tok, bf(out_w), bf(krp.reshape(8, 128, 96)),
                    bf(kgp.reshape(8, 128, 96)))
    lr = o[0].reshape(8, 16, 12, 8).transpose(2, 0, 1, 3)
    lg = o[1].reshape(8, 16, 12, 8).transpose(2, 0, 1, 3)
    return jnp.concatenate([lr, lg], axis=-1)
